# baseline (speedup 1.0000x reference)
import functools
import math

import jax
import jax.numpy as jnp
from jax.experimental import pallas as pl

N_RES = 16384
N_ATOMS = 131072
PAD = 32
ACD = 16
ATD = 8
RCD = 32
RID = 32
MAXSEQ = 2048
AC_VOCAB = 38
AT_VOCAB = 6
RC_VOCAB = 22
MUL_OUT_V = 64
SCAL_OUT = 256
SCAL_IN = PAD * ACD + PAD * ATD + RCD + RID

_B = 256
_AC_W = PAD * AC_VOCAB
_AT_W = PAD * AT_VOCAB


def _embed_kernel(codes_ref, types_ref, mask_ref, rel0_ref, rel1_ref, rel2_ref,
                  rc_ref, ri_ref, ta_ref, tt_ref, trc_ref, tri_ref, wv_ref,
                  vec_ref, scal_ref):
    mask = mask_ref[...]
    codes = codes_ref[...]
    types = types_ref[...]

    iota_ac = jax.lax.broadcasted_iota(jnp.int32, (_B, AC_VOCAB), 1)
    iota_at = jax.lax.broadcasted_iota(jnp.int32, (_B, AT_VOCAB), 1)
    ac_parts = []
    at_parts = []
    for p in range(PAD):
        m = mask[:, p:p + 1]
        ac_parts.append((iota_ac == codes[:, p:p + 1]).astype(jnp.float32) * m)
        at_parts.append((iota_at == types[:, p:p + 1]).astype(jnp.float32) * m)
    oh_ac = jnp.concatenate(ac_parts, axis=1)
    oh_at = jnp.concatenate(at_parts, axis=1)

    oh_rc = (jax.lax.broadcasted_iota(jnp.int32, (_B, RC_VOCAB), 1)
             == rc_ref[...]).astype(jnp.float32)
    oh_ri = (jax.lax.broadcasted_iota(jnp.int32, (_B, MAXSEQ), 1)
             == ri_ref[...]).astype(jnp.float32)

    scal = jnp.dot(oh_ac, ta_ref[...], preferred_element_type=jnp.float32)
    scal += jnp.dot(oh_at, tt_ref[...], preferred_element_type=jnp.float32)
    scal += jnp.dot(oh_rc, trc_ref[...], preferred_element_type=jnp.float32)
    scal += jnp.dot(oh_ri, tri_ref[...], preferred_element_type=jnp.float32)
    scal_ref[...] = scal

    wv = wv_ref[...]
    v0 = jnp.dot(rel0_ref[...], wv, preferred_element_type=jnp.float32)
    v1 = jnp.dot(rel1_ref[...], wv, preferred_element_type=jnp.float32)
    v2 = jnp.dot(rel2_ref[...], wv, preferred_element_type=jnp.float32)
    vec_ref[...] = jnp.concatenate([v0, v1, v2], axis=1)


@functools.partial(jax.jit, donate_argnums=())
def kernel(residue_base_coords, residue_index, residue_index_atomwise,
           residue_relative_coords, atom_code_index, atom_type_index,
           residue_code_index, residue_sequence_index,
           atom_code_table, atom_type_table, residue_code_table,
           residue_index_table, W_vec, W_scal):
    rid = residue_index_atomwise
    res_ids = jnp.arange(N_RES, dtype=rid.dtype)
    starts = jnp.searchsorted(rid, res_ids, side='left').astype(jnp.int32)
    ends = jnp.searchsorted(rid, res_ids, side='right').astype(jnp.int32)
    counts = ends - starts

    slot = jnp.arange(PAD, dtype=jnp.int32)
    gidx = jnp.minimum(starts[:, None] + slot[None, :], N_ATOMS - 1)
    mask = (slot[None, :] < counts[:, None])
    maskf = mask.astype(jnp.float32)

    codes_p = jnp.take(atom_code_index, gidx, axis=0)
    types_p = jnp.take(atom_type_index, gidx, axis=0)
    rel_p = jnp.take(residue_relative_coords, gidx.reshape(-1), axis=0)
    rel_p = rel_p.reshape(N_RES, PAD, 3) * maskf[..., None]
    rel0, rel1, rel2 = rel_p[..., 0], rel_p[..., 1], rel_p[..., 2]

    inv_s = 1.0 / math.sqrt(float(SCAL_IN))
    wa = W_scal[:PAD * ACD].reshape(PAD, ACD, SCAL_OUT)
    ta = jnp.einsum('cd,pdo->pco', atom_code_table, wa).reshape(_AC_W, SCAL_OUT) * inv_s
    wt = W_scal[PAD * ACD:PAD * ACD + PAD * ATD].reshape(PAD, ATD, SCAL_OUT)
    tt = jnp.einsum('cd,pdo->pco', atom_type_table, wt).reshape(_AT_W, SCAL_OUT) * inv_s
    off = PAD * ACD + PAD * ATD
    trc = residue_code_table @ W_scal[off:off + RCD] * inv_s
    tri = residue_index_table @ W_scal[off + RCD:] * inv_s
    wv = W_vec / math.sqrt(float(PAD))

    rc2 = residue_code_index.reshape(N_RES, 1).astype(jnp.int32)
    ri2 = residue_sequence_index.reshape(N_RES, 1).astype(jnp.int32)

    grid = (N_RES // _B,)
    row = lambda i: (i, 0)
    rep = lambda i: (0, 0)
    vec3, out_scal = pl.pallas_call(
        _embed_kernel,
        grid=grid,
        in_specs=[
            pl.BlockSpec((_B, PAD), row),
            pl.BlockSpec((_B, PAD), row),
            pl.BlockSpec((_B, PAD), row),
            pl.BlockSpec((_B, PAD), row),
            pl.BlockSpec((_B, PAD), row),
            pl.BlockSpec((_B, PAD), row),
            pl.BlockSpec((_B, 1), row),
            pl.BlockSpec((_B, 1), row),
            pl.BlockSpec((_AC_W, SCAL_OUT), rep),
            pl.BlockSpec((_AT_W, SCAL_OUT), rep),
            pl.BlockSpec((RC_VOCAB, SCAL_OUT), rep),
            pl.BlockSpec((MAXSEQ, SCAL_OUT), rep),
            pl.BlockSpec((PAD, MUL_OUT_V), rep),
        ],
        out_specs=[
            pl.BlockSpec((_B, 3 * MUL_OUT_V), row),
            pl.BlockSpec((_B, SCAL_OUT), row),
        ],
        out_shape=[
            jax.ShapeDtypeStruct((N_RES, 3 * MUL_OUT_V), jnp.float32),
            jax.ShapeDtypeStruct((N_RES, SCAL_OUT), jnp.float32),
        ],
    )(codes_p, types_p, maskf, rel0, rel1, rel2, rc2, ri2,
      ta, tt, trc, tri, wv)

    out_vec = vec3.reshape(N_RES, 3, MUL_OUT_V).transpose(0, 2, 1).reshape(N_RES, -1)
    features = jnp.concatenate([out_vec, out_scal], axis=-1)
    return (residue_base_coords, features)


# speedup vs baseline: 1.0074x; 1.0074x over previous
import functools
import math

import jax
import jax.numpy as jnp
from jax.experimental import pallas as pl

N_RES = 16384
N_ATOMS = 131072
PAD = 32
ACD = 16
ATD = 8
RCD = 32
RID = 32
MAXSEQ = 2048
AC_VOCAB = 38
AT_VOCAB = 6
RC_VOCAB = 22
MUL_OUT_V = 64
SCAL_OUT = 256
SCAL_IN = PAD * ACD + PAD * ATD + RCD + RID

_B = 256
_CT_W = 64


def _embed_kernel(codes_ref, types_ref, mask_ref, rel0_ref, rel1_ref, rel2_ref,
                  rc_ref, ri_ref, ta_ref, trc_ref, tri_ref, wv_ref,
                  vec_ref, scal_ref):
    mask = mask_ref[...]
    codes = codes_ref[...]
    types = types_ref[...]

    oh_rc = (jax.lax.broadcasted_iota(jnp.int32, (_B, RC_VOCAB), 1)
             == rc_ref[...]).astype(jnp.float32)
    oh_ri = (jax.lax.broadcasted_iota(jnp.int32, (_B, MAXSEQ), 1)
             == ri_ref[...]).astype(jnp.float32)
    scal = jnp.dot(oh_rc, trc_ref[...], preferred_element_type=jnp.float32)
    scal += jnp.dot(oh_ri, tri_ref[...], preferred_element_type=jnp.float32)

    iota64 = jax.lax.broadcasted_iota(jnp.int32, (_B, _CT_W), 1)
    for p in range(PAD):
        m = mask[:, p:p + 1]
        oh = jnp.logical_or(iota64 == codes[:, p:p + 1],
                            iota64 == types[:, p:p + 1] + AC_VOCAB)
        oh = oh.astype(jnp.float32) * m
        scal += jnp.dot(oh, ta_ref[p * _CT_W:(p + 1) * _CT_W, :],
                        preferred_element_type=jnp.float32)
    scal_ref[...] = scal

    wv = wv_ref[...]
    v0 = jnp.dot(rel0_ref[...], wv, preferred_element_type=jnp.float32)
    v1 = jnp.dot(rel1_ref[...], wv, preferred_element_type=jnp.float32)
    v2 = jnp.dot(rel2_ref[...], wv, preferred_element_type=jnp.float32)
    vec_ref[...] = jnp.concatenate([v0, v1, v2], axis=1)


@functools.partial(jax.jit, donate_argnums=())
def kernel(residue_base_coords, residue_index, residue_index_atomwise,
           residue_relative_coords, atom_code_index, atom_type_index,
           residue_code_index, residue_sequence_index,
           atom_code_table, atom_type_table, residue_code_table,
           residue_index_table, W_vec, W_scal):
    rid = residue_index_atomwise
    res_ids = jnp.arange(N_RES, dtype=rid.dtype)
    starts = jnp.searchsorted(rid, res_ids, side='left').astype(jnp.int32)
    ends = jnp.searchsorted(rid, res_ids, side='right').astype(jnp.int32)
    counts = ends - starts

    slot = jnp.arange(PAD, dtype=jnp.int32)
    gidx = jnp.minimum(starts[:, None] + slot[None, :], N_ATOMS - 1)
    mask = (slot[None, :] < counts[:, None])
    maskf = mask.astype(jnp.float32)

    codes_p = jnp.take(atom_code_index, gidx, axis=0)
    types_p = jnp.take(atom_type_index, gidx, axis=0)
    rel_p = jnp.take(residue_relative_coords, gidx.reshape(-1), axis=0)
    rel_p = rel_p.reshape(N_RES, PAD, 3) * maskf[..., None]
    rel0, rel1, rel2 = rel_p[..., 0], rel_p[..., 1], rel_p[..., 2]

    inv_s = 1.0 / math.sqrt(float(SCAL_IN))
    wa = W_scal[:PAD * ACD].reshape(PAD, ACD, SCAL_OUT)
    ta_pco = jnp.einsum('cd,pdo->pco', atom_code_table, wa) * inv_s
    wt = W_scal[PAD * ACD:PAD * ACD + PAD * ATD].reshape(PAD, ATD, SCAL_OUT)
    tt_pco = jnp.einsum('cd,pdo->pco', atom_type_table, wt) * inv_s
    pad_rows = jnp.zeros((PAD, _CT_W - AC_VOCAB - AT_VOCAB, SCAL_OUT), jnp.float32)
    ta = jnp.concatenate([ta_pco, tt_pco, pad_rows], axis=1).reshape(PAD * _CT_W, SCAL_OUT)
    off = PAD * ACD + PAD * ATD
    trc = residue_code_table @ W_scal[off:off + RCD] * inv_s
    tri = residue_index_table @ W_scal[off + RCD:] * inv_s
    wv = W_vec / math.sqrt(float(PAD))

    rc2 = residue_code_index.reshape(N_RES, 1).astype(jnp.int32)
    ri2 = residue_sequence_index.reshape(N_RES, 1).astype(jnp.int32)

    grid = (N_RES // _B,)
    row = lambda i: (i, 0)
    rep = lambda i: (0, 0)
    vec3, out_scal = pl.pallas_call(
        _embed_kernel,
        grid=grid,
        in_specs=[
            pl.BlockSpec((_B, PAD), row),
            pl.BlockSpec((_B, PAD), row),
            pl.BlockSpec((_B, PAD), row),
            pl.BlockSpec((_B, PAD), row),
            pl.BlockSpec((_B, PAD), row),
            pl.BlockSpec((_B, PAD), row),
            pl.BlockSpec((_B, 1), row),
            pl.BlockSpec((_B, 1), row),
            pl.BlockSpec((PAD * _CT_W, SCAL_OUT), rep),
            pl.BlockSpec((RC_VOCAB, SCAL_OUT), rep),
            pl.BlockSpec((MAXSEQ, SCAL_OUT), rep),
            pl.BlockSpec((PAD, MUL_OUT_V), rep),
        ],
        out_specs=[
            pl.BlockSpec((_B, 3 * MUL_OUT_V), row),
            pl.BlockSpec((_B, SCAL_OUT), row),
        ],
        out_shape=[
            jax.ShapeDtypeStruct((N_RES, 3 * MUL_OUT_V), jnp.float32),
            jax.ShapeDtypeStruct((N_RES, SCAL_OUT), jnp.float32),
        ],
    )(codes_p, types_p, maskf, rel0, rel1, rel2, rc2, ri2,
      ta, trc, tri, wv)

    out_vec = vec3.reshape(N_RES, 3, MUL_OUT_V).transpose(0, 2, 1).reshape(N_RES, -1)
    features = jnp.concatenate([out_vec, out_scal], axis=-1)
    return (residue_base_coords, features)
